# Initial kernel scaffold; baseline (speedup 1.0000x reference)
import jax
import jax.numpy as jnp
from jax.experimental import pallas as pl


def kernel(value, sampling_locations, attention_weights):
    raise NotImplementedError("write your pallas kernel here")


# speedup vs baseline: 2.3759x; 2.3759x over previous
import functools

import jax
import jax.numpy as jnp
from jax import lax
from jax.experimental import pallas as pl
from jax.experimental.pallas import tpu as pltpu
from jax.experimental.pallas import tpu_sc as plsc

H_SP, W_SP = 100, 100
NC, NS = 2, 16
NW = NC * NS
LANES = 16


def kernel(value, sampling_locations, attention_weights):
    B, Lin, nH, D = value.shape
    _, Lq, _, L, P, _ = sampling_locations.shape
    T = L * P * 4
    NP = B * Lq
    PPW = NP // NW
    NBQ = 64
    NBLK = PPW // NBQ
    WPB = Lq // PPW

    table = value.reshape(B * Lin * nH, D)
    x = sampling_locations[..., 0].reshape(B, Lq, nH, L * P)
    y = sampling_locations[..., 1].reshape(B, Lq, nH, L * P)
    a = attention_weights.reshape(B, Lq, nH, L * P)
    rec = jnp.concatenate([x, y, a, jnp.zeros_like(a)], axis=-1)
    rec = rec.reshape(NP * nH * T)
    RB = nH * T

    mesh = plsc.VectorSubcoreMesh(
        core_axis_name="c", subcore_axis_name="s", num_cores=NC, num_subcores=NS
    )

    @functools.partial(
        pl.kernel,
        out_type=jax.ShapeDtypeStruct((NP, nH * D), jnp.float32),
        mesh=mesh,
        scratch_types=[
            pltpu.VMEM((NBQ * nH * T,), jnp.float32),
            pltpu.VMEM((NBQ, nH * T), jnp.int32),
            pltpu.VMEM((NBQ * nH * T,), jnp.float32),
            pltpu.VMEM((nH * T, D), jnp.float32),
            pltpu.VMEM((NBQ, nH * D), jnp.float32),
            pltpu.SemaphoreType.DMA,
        ],
        compiler_params=pltpu.CompilerParams(
            needs_layout_passes=False, use_tc_tiling_on_sc=False
        ),
    )
    def sc_kernel(table_hbm, rec_hbm, out_hbm, rec_v, idx_v, w_v, rows_v, out_v, sem):
        wid = lax.axis_index("c") * NS + lax.axis_index("s")
        b = wid // WPB
        vb0 = b * (Lin * nH)
        lane = lax.iota(jnp.int32, LANES)
        perm_p = lane & 3
        csel_x = (lane >> 2) & 1
        csel_y = lane >> 3
        fone = jnp.full((LANES,), 1.0, jnp.float32)

        @pl.loop(0, NBLK)
        def _block(j):
            base = wid * PPW + j * NBQ
            pltpu.sync_copy(rec_hbm.at[pl.ds(base * RB, NBQ * RB)], rec_v)

            @pl.loop(0, NBQ)
            def _pairs(p):
                poff = jnp.zeros((LANES,), jnp.int32) + p * RB
                for h in range(nH):
                    xg = plsc.load_gather(rec_v, [poff + (h * T + perm_p)])
                    yg = plsc.load_gather(rec_v, [poff + (h * T + 4 + perm_p)])
                    ag = plsc.load_gather(rec_v, [poff + (h * T + 8 + perm_p)])
                    fx = xg * jnp.float32(W_SP) - jnp.float32(0.5)
                    fy = yg * jnp.float32(H_SP) - jnp.float32(0.5)
                    tx = fx.astype(jnp.int32)
                    ty = fy.astype(jnp.int32)
                    x0 = jnp.where(fx < tx.astype(jnp.float32), tx - 1, tx)
                    y0 = jnp.where(fy < ty.astype(jnp.float32), ty - 1, ty)
                    lx = fx - x0.astype(jnp.float32)
                    ly = fy - y0.astype(jnp.float32)
                    xc = x0 + csel_x
                    yc = y0 + csel_y
                    wx = jnp.where(csel_x == 1, lx, fone - lx)
                    wy = jnp.where(csel_y == 1, ly, fone - ly)
                    ok = (xc >= 0) & (xc <= W_SP - 1) & (yc >= 0) & (yc <= H_SP - 1)
                    w = jnp.where(ok, ag * wx * wy, jnp.zeros((LANES,), jnp.float32))
                    cx = jnp.minimum(jnp.maximum(xc, 0), W_SP - 1)
                    cy = jnp.minimum(jnp.maximum(yc, 0), H_SP - 1)
                    idx = (vb0 + h) + (cy * W_SP + cx) * nH
                    idx_v[p, pl.ds(h * T, LANES)] = idx
                    w_v[pl.ds(p * RB + h * T, LANES)] = w

            @pl.loop(0, NBQ)
            def _accum(p):
                pltpu.async_copy(table_hbm.at[idx_v.at[p]], rows_v, sem).wait()
                poff = jnp.zeros((LANES,), jnp.int32) + p * RB
                for h in range(nH):
                    acc_lo = jnp.zeros((LANES,), jnp.float32)
                    acc_hi = jnp.zeros((LANES,), jnp.float32)
                    for k in range(T):
                        r = h * T + k
                        wk = plsc.load_gather(w_v, [poff + r])
                        acc_lo = acc_lo + wk * rows_v[r, pl.ds(0, LANES)]
                        acc_hi = acc_hi + wk * rows_v[r, pl.ds(LANES, LANES)]
                    out_v[p, pl.ds(h * D, LANES)] = acc_lo
                    out_v[p, pl.ds(h * D + LANES, LANES)] = acc_hi

            pltpu.sync_copy(out_v, out_hbm.at[pl.ds(base, NBQ)])

    out = sc_kernel(table, rec)
    return out.reshape(B, Lq, nH * D)


# speedup vs baseline: 3.0019x; 1.2635x over previous
import functools

import jax
import jax.numpy as jnp
from jax import lax
from jax.experimental import pallas as pl
from jax.experimental.pallas import tpu as pltpu
from jax.experimental.pallas import tpu_sc as plsc

H_SP, W_SP = 100, 100
NC, NS = 2, 16
NW = NC * NS
LANES = 16


def kernel(value, sampling_locations, attention_weights):
    B, Lin, nH, D = value.shape
    _, Lq, _, L, P, _ = sampling_locations.shape
    T = L * P * 4
    NP = B * Lq
    PPW = NP // NW
    NBQ = 64
    NBLK = PPW // NBQ
    WPB = Lq // PPW

    table = value.reshape(B * Lin * nH, D)
    x = sampling_locations[..., 0].reshape(B, Lq, nH, L * P)
    y = sampling_locations[..., 1].reshape(B, Lq, nH, L * P)
    a = attention_weights.reshape(B, Lq, nH, L * P)
    rec = jnp.concatenate([x, y, a, jnp.zeros_like(a)], axis=-1)
    rec = rec.reshape(NP * nH * T)
    RB = nH * T

    mesh = plsc.VectorSubcoreMesh(
        core_axis_name="c", subcore_axis_name="s", num_cores=NC, num_subcores=NS
    )

    @functools.partial(
        pl.kernel,
        out_type=jax.ShapeDtypeStruct((NP, nH * D), jnp.float32),
        mesh=mesh,
        scratch_types=[
            pltpu.VMEM((NBQ * nH * T,), jnp.float32),
            pltpu.VMEM((NBQ, nH * T), jnp.int32),
            pltpu.VMEM((NBQ * nH * T,), jnp.float32),
            pltpu.VMEM((4, nH * T, D), jnp.float32),
            pltpu.VMEM((NBQ, nH * D), jnp.float32),
            pltpu.SemaphoreType.DMA,
            pltpu.SemaphoreType.DMA,
            pltpu.SemaphoreType.DMA,
            pltpu.SemaphoreType.DMA,
        ],
        compiler_params=pltpu.CompilerParams(
            needs_layout_passes=False, use_tc_tiling_on_sc=False
        ),
    )
    def sc_kernel(
        table_hbm, rec_hbm, out_hbm, rec_v, idx_v, w_v, rows_v, out_v, s0, s1, s2, s3
    ):
        sems = (s0, s1, s2, s3)
        NBUF = 4
        wid = lax.axis_index("c") * NS + lax.axis_index("s")
        b = wid // WPB
        vb0 = b * (Lin * nH)
        lane = lax.iota(jnp.int32, LANES)
        perm_p = lane & 3
        csel_x = (lane >> 2) & 1
        csel_y = lane >> 3
        fone = jnp.full((LANES,), 1.0, jnp.float32)

        @pl.loop(0, NBLK)
        def _block(j):
            base = wid * PPW + j * NBQ
            pltpu.sync_copy(rec_hbm.at[pl.ds(base * RB, NBQ * RB)], rec_v)

            @pl.loop(0, NBQ)
            def _pairs(p):
                poff = jnp.zeros((LANES,), jnp.int32) + p * RB
                for h in range(nH):
                    xg = plsc.load_gather(rec_v, [poff + (h * T + perm_p)])
                    yg = plsc.load_gather(rec_v, [poff + (h * T + 4 + perm_p)])
                    ag = plsc.load_gather(rec_v, [poff + (h * T + 8 + perm_p)])
                    fx = xg * jnp.float32(W_SP) - jnp.float32(0.5)
                    fy = yg * jnp.float32(H_SP) - jnp.float32(0.5)
                    tx = fx.astype(jnp.int32)
                    ty = fy.astype(jnp.int32)
                    x0 = jnp.where(fx < tx.astype(jnp.float32), tx - 1, tx)
                    y0 = jnp.where(fy < ty.astype(jnp.float32), ty - 1, ty)
                    lx = fx - x0.astype(jnp.float32)
                    ly = fy - y0.astype(jnp.float32)
                    xc = x0 + csel_x
                    yc = y0 + csel_y
                    wx = jnp.where(csel_x == 1, lx, fone - lx)
                    wy = jnp.where(csel_y == 1, ly, fone - ly)
                    ok = (xc >= 0) & (xc <= W_SP - 1) & (yc >= 0) & (yc <= H_SP - 1)
                    w = jnp.where(ok, ag * wx * wy, jnp.zeros((LANES,), jnp.float32))
                    cx = jnp.minimum(jnp.maximum(xc, 0), W_SP - 1)
                    cy = jnp.minimum(jnp.maximum(yc, 0), H_SP - 1)
                    idx = (vb0 + h) + (cy * W_SP + cx) * nH
                    idx_v[p, pl.ds(h * T, LANES)] = idx
                    w_v[pl.ds(p * RB + h * T, LANES)] = w

            for i in range(NBUF):
                pltpu.async_copy(table_hbm.at[idx_v.at[i]], rows_v.at[i], sems[i])

            @pl.loop(0, NBQ, step=NBUF)
            def _accum(p0):
                for i in range(NBUF):
                    p = p0 + i
                    pltpu.make_async_copy(
                        table_hbm.at[idx_v.at[p]], rows_v.at[i], sems[i]
                    ).wait()
                    poff = jnp.zeros((LANES,), jnp.int32) + p * RB
                    for h in range(nH):
                        acc_lo = jnp.zeros((LANES,), jnp.float32)
                        acc_hi = jnp.zeros((LANES,), jnp.float32)
                        for k in range(T):
                            r = h * T + k
                            wk = plsc.load_gather(w_v, [poff + r])
                            acc_lo = acc_lo + wk * rows_v[i, r, pl.ds(0, LANES)]
                            acc_hi = acc_hi + wk * rows_v[i, r, pl.ds(LANES, LANES)]
                        out_v[p, pl.ds(h * D, LANES)] = acc_lo
                        out_v[p, pl.ds(h * D + LANES, LANES)] = acc_hi
                    nxt = jnp.minimum(p + NBUF, NBQ - 1)
                    pltpu.async_copy(table_hbm.at[idx_v.at[nxt]], rows_v.at[i], sems[i])

            for i in range(NBUF):
                pltpu.make_async_copy(
                    table_hbm.at[idx_v.at[0]], rows_v.at[i], sems[i]
                ).wait()

            pltpu.sync_copy(out_v, out_hbm.at[pl.ds(base, NBQ)])

    out = sc_kernel(table, rec)
    return out.reshape(B, Lq, nH * D)


# speedup vs baseline: 3.4137x; 1.1372x over previous
import functools

import jax
import jax.numpy as jnp
from jax import lax
from jax.experimental import pallas as pl
from jax.experimental.pallas import tpu as pltpu
from jax.experimental.pallas import tpu_sc as plsc

H_SP, W_SP = 100, 100
NC, NS = 2, 16
NW = NC * NS
LANES = 16


def kernel(value, sampling_locations, attention_weights):
    B, Lin, nH, D = value.shape
    _, Lq, _, L, P, _ = sampling_locations.shape
    T = L * P * 4
    NP = B * Lq
    PPW = NP // NW
    NBQ = 64
    NBLK = PPW // NBQ
    WPB = Lq // PPW

    table = value.reshape(B * Lin * nH, D).astype(jnp.bfloat16)
    x = sampling_locations[..., 0].reshape(B, Lq, nH, L * P)
    y = sampling_locations[..., 1].reshape(B, Lq, nH, L * P)
    a = attention_weights.reshape(B, Lq, nH, L * P)
    rec = jnp.concatenate([x, y, a, jnp.zeros_like(a)], axis=-1)
    rec = rec.reshape(NP * nH * T)
    RB = nH * T

    mesh = plsc.VectorSubcoreMesh(
        core_axis_name="c", subcore_axis_name="s", num_cores=NC, num_subcores=NS
    )

    @functools.partial(
        pl.kernel,
        out_type=jax.ShapeDtypeStruct((NP, nH * D), jnp.float32),
        mesh=mesh,
        scratch_types=[
            pltpu.VMEM((NBQ * nH * T,), jnp.float32),
            pltpu.VMEM((NBQ, nH * T), jnp.int32),
            pltpu.VMEM((NBQ * nH * T,), jnp.float32),
            pltpu.VMEM((4, nH * T, D), jnp.bfloat16),
            pltpu.VMEM((NBQ, nH * D), jnp.float32),
            pltpu.SemaphoreType.DMA,
            pltpu.SemaphoreType.DMA,
            pltpu.SemaphoreType.DMA,
            pltpu.SemaphoreType.DMA,
        ],
        compiler_params=pltpu.CompilerParams(
            needs_layout_passes=False, use_tc_tiling_on_sc=False
        ),
    )
    def sc_kernel(
        table_hbm, rec_hbm, out_hbm, rec_v, idx_v, w_v, rows_v, out_v, s0, s1, s2, s3
    ):
        sems = (s0, s1, s2, s3)
        NBUF = 4
        wid = lax.axis_index("c") * NS + lax.axis_index("s")
        b = wid // WPB
        vb0 = b * (Lin * nH)
        lane = lax.iota(jnp.int32, LANES)
        perm_p = lane & 3
        csel_x = (lane >> 2) & 1
        csel_y = lane >> 3
        fone = jnp.full((LANES,), 1.0, jnp.float32)

        @pl.loop(0, NBLK)
        def _block(j):
            base = wid * PPW + j * NBQ
            pltpu.sync_copy(rec_hbm.at[pl.ds(base * RB, NBQ * RB)], rec_v)

            @pl.loop(0, NBQ)
            def _pairs(p):
                poff = jnp.zeros((LANES,), jnp.int32) + p * RB
                for h in range(nH):
                    xg = plsc.load_gather(rec_v, [poff + (h * T + perm_p)])
                    yg = plsc.load_gather(rec_v, [poff + (h * T + 4 + perm_p)])
                    ag = plsc.load_gather(rec_v, [poff + (h * T + 8 + perm_p)])
                    fx = xg * jnp.float32(W_SP) - jnp.float32(0.5)
                    fy = yg * jnp.float32(H_SP) - jnp.float32(0.5)
                    tx = fx.astype(jnp.int32)
                    ty = fy.astype(jnp.int32)
                    x0 = jnp.where(fx < tx.astype(jnp.float32), tx - 1, tx)
                    y0 = jnp.where(fy < ty.astype(jnp.float32), ty - 1, ty)
                    lx = fx - x0.astype(jnp.float32)
                    ly = fy - y0.astype(jnp.float32)
                    xc = x0 + csel_x
                    yc = y0 + csel_y
                    wx = jnp.where(csel_x == 1, lx, fone - lx)
                    wy = jnp.where(csel_y == 1, ly, fone - ly)
                    ok = (xc >= 0) & (xc <= W_SP - 1) & (yc >= 0) & (yc <= H_SP - 1)
                    w = jnp.where(ok, ag * wx * wy, jnp.zeros((LANES,), jnp.float32))
                    cx = jnp.minimum(jnp.maximum(xc, 0), W_SP - 1)
                    cy = jnp.minimum(jnp.maximum(yc, 0), H_SP - 1)
                    idx = (vb0 + h) + (cy * W_SP + cx) * nH
                    idx_v[p, pl.ds(h * T, LANES)] = idx
                    w_v[pl.ds(p * RB + h * T, LANES)] = w

            for i in range(NBUF):
                pltpu.async_copy(table_hbm.at[idx_v.at[i]], rows_v.at[i], sems[i])

            @pl.loop(0, NBQ, step=NBUF)
            def _accum(p0):
                for i in range(NBUF):
                    p = p0 + i
                    pltpu.make_async_copy(
                        table_hbm.at[idx_v.at[p]], rows_v.at[i], sems[i]
                    ).wait()
                    poff = jnp.zeros((LANES,), jnp.int32) + p * RB
                    for h in range(nH):
                        acc_ev = jnp.zeros((LANES,), jnp.float32)
                        acc_od = jnp.zeros((LANES,), jnp.float32)
                        for k in range(T):
                            r = h * T + k
                            wk = plsc.load_gather(w_v, [poff + r])
                            ev, od = plsc.unpack(
                                rows_v[i, r], format=plsc.PackFormat.INTERLEAVED
                            )
                            acc_ev = acc_ev + wk * ev
                            acc_od = acc_od + wk * od
                        out_v[p, pl.ds(h * D, LANES)] = acc_ev
                        out_v[p, pl.ds(h * D + LANES, LANES)] = acc_od
                    nxt = jnp.minimum(p + NBUF, NBQ - 1)
                    pltpu.async_copy(table_hbm.at[idx_v.at[nxt]], rows_v.at[i], sems[i])

            for i in range(NBUF):
                pltpu.make_async_copy(
                    table_hbm.at[idx_v.at[0]], rows_v.at[i], sems[i]
                ).wait()

            pltpu.sync_copy(out_v, out_hbm.at[pl.ds(base, NBQ)])

    out = sc_kernel(table, rec)
    out = out.reshape(B, Lq, nH, 2, D // 2).transpose(0, 1, 2, 4, 3)
    return out.reshape(B, Lq, nH * D)
